# baseline (speedup 1.0000x reference)
import jax
import jax.numpy as jnp
from jax.experimental import pallas as pl
from jax.experimental.pallas import tpu as pltpu


def kernel(x, pos_emb):
    batch = x.shape[0]
    max_len, d_model = pos_emb.shape
    block_rows = 1024
    n_blocks = max_len // block_rows

    def body(p_ref, o_ref, sems):
        i = pl.program_id(0)
        copies = [
            pltpu.make_async_copy(
                p_ref,
                o_ref.at[b, pl.ds(i * block_rows, block_rows), :],
                sems.at[b],
            )
            for b in range(batch)
        ]
        for c in copies:
            c.start()
        for c in copies:
            c.wait()

    return pl.pallas_call(
        body,
        grid=(n_blocks,),
        in_specs=[pl.BlockSpec((block_rows, d_model), lambda i: (i, 0))],
        out_specs=pl.BlockSpec(memory_space=pl.ANY),
        out_shape=jax.ShapeDtypeStruct((batch, max_len, d_model), pos_emb.dtype),
        scratch_shapes=[pltpu.SemaphoreType.DMA((batch,))],
    )(pos_emb)
